# baseline (speedup 1.0000x reference)
import jax
import jax.numpy as jnp
from jax.experimental import pallas as pl
from jax.experimental.pallas import tpu as pltpu

_NN = 500
_QT = 64
_KP = 102400
_NC = 8
_INT_MIN = -2147483648


def _f32_key(d2):
    u = jax.lax.bitcast_convert_type(d2, jnp.int32)
    return jnp.where(u >= 0, u, _INT_MIN - u)


def _key_f32(k):
    u = jnp.where(k >= 0, k, _INT_MIN - k)
    return jax.lax.bitcast_convert_type(u, jnp.float32)


def _lwr_body(x_ref, xt_ref, kn_ref, yn_ref, out_ref, keys_ref):
    qt = x_ref.shape[0]
    kp = xt_ref.shape[1]
    chunk = kp // _NC

    x = x_ref[:, :]
    qn = jnp.sum(x * x, axis=1, keepdims=True)

    def build(c, mn):
        xt = xt_ref[:, pl.ds(c * chunk, chunk)]
        kn = kn_ref[:, pl.ds(c * chunk, chunk)]
        prod = jax.lax.dot_general(
            x, xt, (((1,), (0,)), ((), ())),
            preferred_element_type=jnp.float32,
        )
        d2 = qn - 2.0 * prod + kn
        key = _f32_key(d2)
        keys_ref[:, pl.ds(c * chunk, chunk)] = key
        return jnp.minimum(mn, jnp.min(key, axis=1, keepdims=True))

    min0 = jnp.full((qt, 1), 2147483647, jnp.int32)
    lo0 = jax.lax.fori_loop(0, _NC, build, min0, unroll=False)

    hi0 = jnp.max(keys_ref[:, :512], axis=1, keepdims=True)
    rng = jnp.max(hi0.astype(jnp.float32) - lo0.astype(jnp.float32))
    rexp = (jax.lax.bitcast_convert_type(rng, jnp.int32) >> 23) - 127
    n_it = jnp.clip(rexp + 2, 1, 34)

    ones_col = jnp.ones((chunk, 1), jnp.float32)

    def count_le(mid):
        def body(c, acc):
            k = keys_ref[:, pl.ds(c * chunk, chunk)]
            m = (k <= mid).astype(jnp.float32)
            return acc + jax.lax.dot_general(
                m, ones_col, (((1,), (0,)), ((), ())),
                preferred_element_type=jnp.float32)
        return jax.lax.fori_loop(
            0, _NC, body, jnp.zeros((qt, 1), jnp.float32), unroll=8)

    def bisect(i, carry):
        lo, hi = carry
        mid = lo + ((hi - lo) >> 1)
        ge = count_le(mid) >= jnp.float32(_NN)
        return jnp.where(ge, lo, mid + 1), jnp.where(ge, mid, hi)

    _, t_key = jax.lax.fori_loop(0, n_it, bisect, (lo0, hi0),
                                 unroll=False)

    t = _key_f32(t_key)
    tau = jnp.maximum(jnp.maximum(t, 0.0), 1e-8)
    neg_scale = -0.5 / tau

    def wsum(c, carry):
        s_w, s_wy, c_tot, c_eq, s_yeq = carry
        k = keys_ref[:, pl.ds(c * chunk, chunk)]
        y = yn_ref[:, pl.ds(c * chunk, chunk)]
        m = k <= t_key
        nd = jnp.maximum(_key_f32(k), 0.0)
        w = jnp.where(m, jnp.exp(nd * neg_scale), 0.0)
        eq = k == t_key
        s_w += jnp.sum(w, axis=1, keepdims=True)
        s_wy += jnp.sum(w * y, axis=1, keepdims=True)
        c_tot += jnp.sum(m.astype(jnp.int32), axis=1, keepdims=True)
        c_eq += jnp.sum(eq.astype(jnp.int32), axis=1, keepdims=True)
        s_yeq += jnp.sum(jnp.where(eq, y, 0.0), axis=1, keepdims=True)
        return s_w, s_wy, c_tot, c_eq, s_yeq

    z_f = jnp.zeros((qt, 1), jnp.float32)
    z_i = jnp.zeros((qt, 1), jnp.int32)
    s_w, s_wy, c_tot, c_eq, s_yeq = jax.lax.fori_loop(
        0, _NC, wsum, (z_f, z_f, z_i, z_i, z_f), unroll=2)

    extra = (c_tot - _NN).astype(jnp.float32)
    w_t = jnp.exp(jnp.maximum(t, 0.0) * neg_scale)
    y_t = s_yeq / jnp.maximum(c_eq.astype(jnp.float32), 1.0)
    pred = (s_wy - extra * w_t * y_t) / (s_w - extra * w_t)
    out_ref[:, :] = pred


def kernel(X, train_X, train_y):
    q, d = X.shape
    k = train_X.shape[0]
    kn = jnp.sum(train_X * train_X, axis=1)
    pad = _KP - k
    xt = jnp.pad(train_X.T, ((0, 0), (0, pad)))
    knp = jnp.pad(kn, (0, pad), constant_values=jnp.inf)[None, :]
    ynp = jnp.pad(train_y, (0, pad))[None, :]

    grid = q // _QT
    out = pl.pallas_call(
        _lwr_body,
        grid=(grid,),
        in_specs=[
            pl.BlockSpec((_QT, d), lambda i: (i, 0)),
            pl.BlockSpec((d, _KP), lambda i: (0, 0)),
            pl.BlockSpec((1, _KP), lambda i: (0, 0)),
            pl.BlockSpec((1, _KP), lambda i: (0, 0)),
        ],
        out_specs=pl.BlockSpec((_QT, 1), lambda i: (i, 0)),
        out_shape=jax.ShapeDtypeStruct((q, 1), jnp.float32),
        scratch_shapes=[pltpu.VMEM((_QT, _KP), jnp.int32)],
        compiler_params=pltpu.CompilerParams(
            dimension_semantics=("arbitrary",)),
    )(X, xt, knp, ynp)
    return out[:, 0]


# speedup vs baseline: 1.2261x; 1.2261x over previous
import jax
import jax.numpy as jnp
from jax.experimental import pallas as pl
from jax.experimental.pallas import tpu as pltpu

_NN = 500
_QT = 64
_KP = 102400
_NC = 8
_INT_MIN = -2147483648


def _f32_key(d2):
    u = jax.lax.bitcast_convert_type(d2, jnp.int32)
    return jnp.where(u >= 0, u, _INT_MIN - u)


def _key_f32(k):
    u = jnp.where(k >= 0, k, _INT_MIN - k)
    return jax.lax.bitcast_convert_type(u, jnp.float32)


def _lwr_body(x_ref, xt_ref, kn_ref, yn_ref, out_ref, keys_ref):
    qt = x_ref.shape[0]
    kp = xt_ref.shape[1]
    chunk = kp // _NC

    x = x_ref[:, :]
    qn = jnp.sum(x * x, axis=1, keepdims=True)

    def build(c, mn):
        xt = xt_ref[:, pl.ds(c * chunk, chunk)]
        kn = kn_ref[:, pl.ds(c * chunk, chunk)]
        prod = jax.lax.dot_general(
            x, xt, (((1,), (0,)), ((), ())),
            preferred_element_type=jnp.float32,
        )
        d2 = qn - 2.0 * prod + kn
        key = _f32_key(d2)
        keys_ref[:, pl.ds(c * chunk, chunk)] = key
        return jnp.minimum(mn, jnp.min(key, axis=1, keepdims=True))

    min0 = jnp.full((qt, 1), 2147483647, jnp.int32)
    lo0 = jax.lax.fori_loop(0, _NC, build, min0, unroll=False)

    hi0 = jnp.max(keys_ref[:, :512], axis=1, keepdims=True)
    rng = jnp.max(hi0.astype(jnp.float32) - lo0.astype(jnp.float32))
    rexp = (jax.lax.bitcast_convert_type(rng, jnp.int32) >> 23) - 127
    n_it = jnp.clip(rexp + 2, 1, 34)

    def count_le(mid):
        def body(c, acc):
            k = keys_ref[:, pl.ds(c * chunk, chunk)]
            return acc + jnp.sum((k <= mid).astype(jnp.int32), axis=1,
                                 keepdims=True)
        return jax.lax.fori_loop(
            0, _NC, body, jnp.zeros((qt, 1), jnp.int32), unroll=8)

    def bisect(i, carry):
        lo, hi = carry
        mid = lo + ((hi - lo) >> 1)
        ge = count_le(mid) >= _NN
        return jnp.where(ge, lo, mid + 1), jnp.where(ge, mid, hi)

    _, t_key = jax.lax.fori_loop(0, n_it, bisect, (lo0, hi0),
                                 unroll=False)

    t = _key_f32(t_key)
    tau = jnp.maximum(jnp.maximum(t, 0.0), 1e-8)
    neg_scale = -0.5 / tau

    def wsum(c, carry):
        s_w, s_wy, c_tot, c_eq, s_yeq = carry
        k = keys_ref[:, pl.ds(c * chunk, chunk)]
        y = yn_ref[:, pl.ds(c * chunk, chunk)]
        m = k <= t_key
        nd = jnp.maximum(_key_f32(k), 0.0)
        w = jnp.where(m, jnp.exp(nd * neg_scale), 0.0)
        eq = k == t_key
        s_w += jnp.sum(w, axis=1, keepdims=True)
        s_wy += jnp.sum(w * y, axis=1, keepdims=True)
        c_tot += jnp.sum(m.astype(jnp.int32), axis=1, keepdims=True)
        c_eq += jnp.sum(eq.astype(jnp.int32), axis=1, keepdims=True)
        s_yeq += jnp.sum(jnp.where(eq, y, 0.0), axis=1, keepdims=True)
        return s_w, s_wy, c_tot, c_eq, s_yeq

    z_f = jnp.zeros((qt, 1), jnp.float32)
    z_i = jnp.zeros((qt, 1), jnp.int32)
    s_w, s_wy, c_tot, c_eq, s_yeq = jax.lax.fori_loop(
        0, _NC, wsum, (z_f, z_f, z_i, z_i, z_f), unroll=2)

    extra = (c_tot - _NN).astype(jnp.float32)
    w_t = jnp.exp(jnp.maximum(t, 0.0) * neg_scale)
    y_t = s_yeq / jnp.maximum(c_eq.astype(jnp.float32), 1.0)
    pred = (s_wy - extra * w_t * y_t) / (s_w - extra * w_t)
    out_ref[:, :] = pred


def kernel(X, train_X, train_y):
    q, d = X.shape
    k = train_X.shape[0]
    kn = jnp.sum(train_X * train_X, axis=1)
    pad = _KP - k
    xt = jnp.pad(train_X.T, ((0, 0), (0, pad)))
    knp = jnp.pad(kn, (0, pad), constant_values=jnp.inf)[None, :]
    ynp = jnp.pad(train_y, (0, pad))[None, :]

    grid = q // _QT
    out = pl.pallas_call(
        _lwr_body,
        grid=(grid,),
        in_specs=[
            pl.BlockSpec((_QT, d), lambda i: (i, 0)),
            pl.BlockSpec((d, _KP), lambda i: (0, 0)),
            pl.BlockSpec((1, _KP), lambda i: (0, 0)),
            pl.BlockSpec((1, _KP), lambda i: (0, 0)),
        ],
        out_specs=pl.BlockSpec((_QT, 1), lambda i: (i, 0)),
        out_shape=jax.ShapeDtypeStruct((q, 1), jnp.float32),
        scratch_shapes=[pltpu.VMEM((_QT, _KP), jnp.int32)],
        compiler_params=pltpu.CompilerParams(
            dimension_semantics=("arbitrary",)),
    )(X, xt, knp, ynp)
    return out[:, 0]
